# baseline (speedup 1.0000x reference)
import functools

import jax
import jax.numpy as jnp
from jax.experimental import pallas as pl
from jax.experimental.pallas import tpu as pltpu

_BLOCK_N = 256
_N_CORES = 2


def _mxu_kernel(idx_ref, tgt_ref, table_ref, logits_ref, nll_ref, *, block_n):
    V = table_ref.shape[0]
    idx = idx_ref[...]
    col = jax.lax.broadcasted_iota(jnp.int32, (block_n, V), 1)
    onehot = (col == idx).astype(jnp.float32)
    logits = jnp.dot(onehot, table_ref[...], preferred_element_type=jnp.float32)
    logits_ref[...] = logits

    m = jnp.max(logits, axis=-1, keepdims=True)
    lse = m + jnp.log(jnp.sum(jnp.exp(logits - m), axis=-1, keepdims=True))
    tgt = tgt_ref[...]
    tgt_logit = jnp.sum(jnp.where(col == tgt, logits, 0.0),
                        axis=-1, keepdims=True)
    nll_ref[...] = lse - tgt_logit


def _bigram_forward(idx, table, targets, *, block_n=_BLOCK_N):
    B, T = idx.shape
    V = table.shape[0]
    N = B * T
    num_blocks = N // block_n
    bpc = num_blocks // _N_CORES

    idx_col = idx.astype(jnp.int32).reshape(N, 1)
    tgt_col = targets.astype(jnp.int32).reshape(N, 1)

    kern = functools.partial(_mxu_kernel, block_n=block_n)

    def _blk(i, j):
        return (i * bpc + j, 0)

    logits_flat, nll = pl.pallas_call(
        kern,
        grid=(_N_CORES, bpc),
        in_specs=[
            pl.BlockSpec((block_n, 1), _blk),
            pl.BlockSpec((block_n, 1), _blk),
            pl.BlockSpec((V, V), lambda i, j: (0, 0)),
        ],
        out_specs=(
            pl.BlockSpec((block_n, V), _blk),
            pl.BlockSpec((block_n, 1), _blk),
        ),
        out_shape=(
            jax.ShapeDtypeStruct((N, V), jnp.float32),
            jax.ShapeDtypeStruct((N, 1), jnp.float32),
        ),
        compiler_params=pltpu.CompilerParams(
            dimension_semantics=("parallel", "arbitrary"),
            vmem_limit_bytes=40 * 1024 * 1024,
        ),
        cost_estimate=pl.CostEstimate(
            flops=2 * N * V * V,
            transcendentals=N * V,
            bytes_accessed=N * V * 4 * 2 + V * V * 4,
        ),
    )(idx_col, tgt_col, table)

    logits = logits_flat.reshape(B, T, V)
    loss = jnp.sum(nll[:, 0]) / N
    return logits, loss


def kernel(idx, table, targets):
    return _bigram_forward(idx, table, targets)
